# baseline (speedup 1.0000x reference)
import functools

import jax
import jax.numpy as jnp
from jax import lax
from jax.experimental import pallas as pl
from jax.experimental.pallas import tpu as pltpu
from jax.experimental.pallas import tpu_sc as plsc

NX, NY, NZ = 432, 496, 1
VOXEL_SIZE = (0.16, 0.16, 4.0)
PC_RANGE = (0.0, -39.68, -3.0, 69.12, 39.68, 1.0)
BATCH = 2
N_VOX, MAX_PTS, C_PT = 16000, 32, 4
C_OUT = 64
C_PAD = 128
EPS = 1e-3

CELLS = NX * NY
TRASH = BATCH * CELLS
R_CANVAS = BATCH * CELLS + 8

NB = 320
NSTEPS = N_VOX // NB

SC_CORES, SC_TILES = 2, 16
ROWS_PER_TILE = CELLS // SC_TILES
ZCH = 248
NZCOPIES = ROWS_PER_TILE // ZCH
PPT = N_VOX // SC_TILES
CHUNK = 128
NCHUNKS = (PPT + CHUNK - 1) // CHUNK
PAD_N = SC_TILES * NCHUNKS * CHUNK

TCOLS = 3456
TBLK = CELLS // TCOLS


def _a1_body(vox_ref, coords_ref, nump_ref, w1t_ref, w23t_ref, w2t_ref,
             w3t_ref, maxh_ref, stats_ref):
  i = pl.program_id(0)
  v = vox_ref[...]
  v2 = v.reshape(NB * MAX_PTS, C_PT)
  h = jnp.dot(v2, w1t_ref[...], preferred_element_type=jnp.float32)
  h = h + jnp.dot(v2[:, 0:3], w23t_ref[...], preferred_element_type=jnp.float32)
  npts = nump_ref[...]
  mean = jnp.sum(v[:, :, 0:3], axis=1) / npts
  cf = coords_ref[...].astype(jnp.float32)
  vx, vy, vz = VOXEL_SIZE
  cenx = cf[:, 3:4] * vx + (vx / 2 + PC_RANGE[0])
  ceny = cf[:, 2:3] * vy + (vy / 2 + PC_RANGE[1])
  cenz = cf[:, 1:2] * vz + (vz / 2 + PC_RANGE[2])
  t = (mean[:, 0:1] * w2t_ref[0:1, :] + mean[:, 1:2] * w2t_ref[1:2, :]
       + mean[:, 2:3] * w2t_ref[2:3, :]
       + cenx * w3t_ref[0:1, :] + ceny * w3t_ref[1:2, :]
       + cenz * w3t_ref[2:3, :])
  h3 = h.reshape(NB, MAX_PTS, C_PAD) - t[:, None, :]
  pidx = lax.broadcasted_iota(jnp.int32, (NB, MAX_PTS), 1).astype(jnp.float32)
  m = (pidx < npts).astype(jnp.float32)
  h3 = h3 * m[:, :, None]
  maxh_ref[...] = jnp.max(h3, axis=1)

  @pl.when(i == 0)
  def _():
    stats_ref[...] = jnp.zeros((8, C_PAD), jnp.float32)

  ssum = jnp.sum(h3, axis=(0, 1))
  ssq = jnp.sum(h3 * h3, axis=(0, 1))
  stats_ref[0:1, :] = stats_ref[0:1, :] + ssum[None, :]
  stats_ref[1:2, :] = stats_ref[1:2, :] + ssq[None, :]


def _a2_body(maxh_ref, scale_ref, shift_ref, pf_ref):
  pf_ref[...] = jnp.maximum(
      maxh_ref[...] * scale_ref[...] + shift_ref[...], 0.0)


def _sc_body(pf_hbm, perm_hbm, dest_hbm, zsrc_hbm, out_hbm,
             zeros_v, perm_v, dest_v, rows_v, sem):
  c = lax.axis_index("c")
  s = lax.axis_index("s")

  pltpu.sync_copy(zsrc_hbm, zeros_v)
  base = c * CELLS + s * ROWS_PER_TILE

  def zero_step(t, _):
    pltpu.sync_copy(zeros_v, out_hbm.at[pl.ds(base + t * ZCH, ZCH), :])
    return 0

  lax.fori_loop(0, NZCOPIES, zero_step, 0)
  plsc.subcore_barrier()

  pltpu.sync_copy(perm_hbm.at[s], perm_v)
  pltpu.sync_copy(dest_hbm.at[s], dest_v)

  lo = c * CELLS
  hi = lo + CELLS
  for j in range(NCHUNKS):
    for k in range(CHUNK // 16):
      dv = dest_v[j, pl.ds(k * 16, 16)]
      owned = (dv >= lo) & (dv < hi)
      dest_v[j, pl.ds(k * 16, 16)] = jnp.where(owned, dv, TRASH)

  for j in range(NCHUNKS):
    pltpu.async_copy(pf_hbm.at[perm_v.at[j]], rows_v, sem).wait()
    pltpu.async_copy(rows_v, out_hbm.at[dest_v.at[j]], sem).wait()


def _tr_body(canvas_ref, out_ref):
  out_ref[...] = jnp.swapaxes(canvas_ref[...], 0, 1)[None, 0:C_OUT, :]


@jax.jit
def kernel(voxels, coords, voxel_num_points, W, gamma, beta):
  pad_c = ((0, 0), (0, C_PAD - C_OUT))
  w1t = jnp.pad(jnp.transpose(W[:, 0:C_PT]), pad_c)
  w2t = jnp.pad(jnp.transpose(W[:, C_PT:C_PT + 3]), pad_c)
  w3t = jnp.pad(jnp.transpose(W[:, C_PT + 3:C_PT + 6]), pad_c)
  w23t = w2t + w3t
  gamma_p = jnp.pad(gamma, (0, C_PAD - C_OUT))
  beta_p = jnp.pad(beta, (0, C_PAD - C_OUT))
  numpf = voxel_num_points.astype(jnp.float32)[:, None]

  maxh, stats = pl.pallas_call(
      _a1_body,
      grid=(NSTEPS,),
      in_specs=[
          pl.BlockSpec((NB, MAX_PTS, C_PT), lambda i: (i, 0, 0)),
          pl.BlockSpec((NB, 4), lambda i: (i, 0)),
          pl.BlockSpec((NB, 1), lambda i: (i, 0)),
          pl.BlockSpec((C_PT, C_PAD), lambda i: (0, 0)),
          pl.BlockSpec((3, C_PAD), lambda i: (0, 0)),
          pl.BlockSpec((3, C_PAD), lambda i: (0, 0)),
          pl.BlockSpec((3, C_PAD), lambda i: (0, 0)),
      ],
      out_specs=[
          pl.BlockSpec((NB, C_PAD), lambda i: (i, 0)),
          pl.BlockSpec((8, C_PAD), lambda i: (0, 0)),
      ],
      out_shape=[
          jax.ShapeDtypeStruct((N_VOX, C_PAD), jnp.float32),
          jax.ShapeDtypeStruct((8, C_PAD), jnp.float32),
      ],
  )(voxels, coords, numpf, w1t, w23t, w2t, w3t)

  cnt = float(N_VOX * MAX_PTS)
  mu = stats[0] / cnt
  var = stats[1] / cnt - mu * mu
  scale = gamma_p * lax.rsqrt(var + EPS)
  shift = beta_p - mu * scale

  pf = pl.pallas_call(
      _a2_body,
      grid=(4,),
      in_specs=[
          pl.BlockSpec((N_VOX // 4, C_PAD), lambda i: (i, 0)),
          pl.BlockSpec((1, C_PAD), lambda i: (0, 0)),
          pl.BlockSpec((1, C_PAD), lambda i: (0, 0)),
      ],
      out_specs=pl.BlockSpec((N_VOX // 4, C_PAD), lambda i: (i, 0)),
      out_shape=jax.ShapeDtypeStruct((N_VOX, C_PAD), jnp.float32),
  )(maxh, scale[None, :], shift[None, :])

  key = coords[:, 0] * CELLS + coords[:, 2] * NX + coords[:, 3]
  sk, perm = lax.sort_key_val(key, jnp.arange(N_VOX, dtype=jnp.int32),
                              is_stable=True)
  is_win = jnp.concatenate(
      [sk[1:] != sk[:-1], jnp.ones((1,), bool)])
  dest = jnp.where(is_win, sk, TRASH)
  pad = PAD_N - N_VOX
  perm_pad = jnp.concatenate(
      [perm, jnp.zeros((pad,), jnp.int32)]).reshape(SC_TILES, NCHUNKS, CHUNK)
  dest_pad = jnp.concatenate(
      [dest, jnp.full((pad,), TRASH, jnp.int32)]).reshape(
          SC_TILES, NCHUNKS, CHUNK)
  zsrc = jnp.zeros((ZCH, C_PAD), jnp.float32)

  mesh = plsc.VectorSubcoreMesh(core_axis_name="c", subcore_axis_name="s",
                                num_cores=SC_CORES, num_subcores=SC_TILES)
  canvas = pl.kernel(
      _sc_body,
      out_type=jax.ShapeDtypeStruct((R_CANVAS, C_PAD), jnp.float32),
      mesh=mesh,
      scratch_types=[
          pltpu.VMEM((ZCH, C_PAD), jnp.float32),
          pltpu.VMEM((NCHUNKS, CHUNK), jnp.int32),
          pltpu.VMEM((NCHUNKS, CHUNK), jnp.int32),
          pltpu.VMEM((CHUNK, C_PAD), jnp.float32),
          pltpu.SemaphoreType.DMA,
      ],
  )(pf, perm_pad, dest_pad, zsrc)

  out = pl.pallas_call(
      _tr_body,
      grid=(BATCH, TBLK),
      in_specs=[pl.BlockSpec((TCOLS, C_PAD), lambda b, j: (b * TBLK + j, 0))],
      out_specs=pl.BlockSpec((1, C_OUT, TCOLS), lambda b, j: (b, 0, j)),
      out_shape=jax.ShapeDtypeStruct((BATCH, C_OUT, CELLS), jnp.float32),
  )(canvas)
  return out.reshape(BATCH, C_OUT * NZ, NY, NX)


# speedup vs baseline: 1.1091x; 1.1091x over previous
import functools

import jax
import jax.numpy as jnp
from jax import lax
from jax.experimental import pallas as pl
from jax.experimental.pallas import tpu as pltpu
from jax.experimental.pallas import tpu_sc as plsc

NX, NY, NZ = 432, 496, 1
VOXEL_SIZE = (0.16, 0.16, 4.0)
PC_RANGE = (0.0, -39.68, -3.0, 69.12, 39.68, 1.0)
BATCH = 2
N_VOX, MAX_PTS, C_PT = 16000, 32, 4
C_OUT = 64
C_PAD = 128
EPS = 1e-3

CELLS = NX * NY
TRASH = BATCH * CELLS
R_CANVAS = BATCH * CELLS + 8

NB = 320
NSTEPS = N_VOX // NB

SC_CORES, SC_TILES = 2, 16
ROWS_PER_TILE = CELLS // SC_TILES
ZCH = 744
NZCOPIES = ROWS_PER_TILE // ZCH
PPT = N_VOX // SC_TILES
CHUNK = 128
NCHUNKS = (PPT + CHUNK - 1) // CHUNK
PAD_N = SC_TILES * NCHUNKS * CHUNK

TCOLS = 3456
TBLK = CELLS // TCOLS


def _a1_body(vox_ref, coords_ref, nump_ref, w1t_ref, w23t_ref, w2t_ref,
             w3t_ref, maxh_ref, stats_ref):
  i = pl.program_id(0)
  v = vox_ref[...]
  v2 = v.reshape(NB * MAX_PTS, C_PT)
  h = jnp.dot(v2, w1t_ref[...], preferred_element_type=jnp.float32)
  h = h + jnp.dot(v2[:, 0:3], w23t_ref[...], preferred_element_type=jnp.float32)
  npts = nump_ref[...]
  mean = jnp.sum(v[:, :, 0:3], axis=1) / npts
  cf = coords_ref[...].astype(jnp.float32)
  vx, vy, vz = VOXEL_SIZE
  cenx = cf[:, 3:4] * vx + (vx / 2 + PC_RANGE[0])
  ceny = cf[:, 2:3] * vy + (vy / 2 + PC_RANGE[1])
  cenz = cf[:, 1:2] * vz + (vz / 2 + PC_RANGE[2])
  t = (mean[:, 0:1] * w2t_ref[0:1, :] + mean[:, 1:2] * w2t_ref[1:2, :]
       + mean[:, 2:3] * w2t_ref[2:3, :]
       + cenx * w3t_ref[0:1, :] + ceny * w3t_ref[1:2, :]
       + cenz * w3t_ref[2:3, :])
  h3 = h.reshape(NB, MAX_PTS, C_PAD) - t[:, None, :]
  pidx = lax.broadcasted_iota(jnp.int32, (NB, MAX_PTS), 1).astype(jnp.float32)
  m = (pidx < npts).astype(jnp.float32)
  h3 = h3 * m[:, :, None]
  maxh_ref[...] = jnp.max(h3, axis=1)

  @pl.when(i == 0)
  def _():
    stats_ref[...] = jnp.zeros((8, C_PAD), jnp.float32)

  ssum = jnp.sum(h3, axis=(0, 1))
  ssq = jnp.sum(h3 * h3, axis=(0, 1))
  stats_ref[0:1, :] = stats_ref[0:1, :] + ssum[None, :]
  stats_ref[1:2, :] = stats_ref[1:2, :] + ssq[None, :]


def _a2_body(maxh_ref, scale_ref, shift_ref, pf_ref):
  pf_ref[...] = jnp.maximum(
      maxh_ref[:, 0:C_OUT] * scale_ref[...] + shift_ref[...], 0.0)


def _sc_body(pf_hbm, perm_hbm, dest_hbm, zsrc_hbm, out_hbm,
             zeros_v, perm_v, dest_v, rows_a, rows_b, sem, sem_g, sem_s):
  c = lax.axis_index("c")
  s = lax.axis_index("s")

  ld_p = pltpu.async_copy(perm_hbm.at[s], perm_v, sem_g)
  ld_d = pltpu.async_copy(dest_hbm.at[s], dest_v, sem_s)

  pltpu.sync_copy(zsrc_hbm, zeros_v)
  base = c * CELLS + s * ROWS_PER_TILE
  zcopies = [
      pltpu.async_copy(zeros_v, out_hbm.at[pl.ds(base + t * ZCH, ZCH), :], sem)
      for t in range(NZCOPIES)
  ]
  ld_p.wait()
  ld_d.wait()

  lo = c * CELLS
  hi = lo + CELLS
  for j in range(NCHUNKS):
    for k in range(CHUNK // 16):
      dv = dest_v[j, pl.ds(k * 16, 16)]
      owned = (dv >= lo) & (dv < hi)
      dest_v[j, pl.ds(k * 16, 16)] = jnp.where(owned, dv, TRASH)

  for cp in zcopies:
    cp.wait()
  plsc.subcore_barrier()

  bufs = [rows_a, rows_b]
  gets = [None] * NCHUNKS
  puts = [None] * NCHUNKS
  gets[0] = pltpu.async_copy(pf_hbm.at[perm_v.at[0]], bufs[0], sem_g)
  gets[1] = pltpu.async_copy(pf_hbm.at[perm_v.at[1]], bufs[1], sem_g)
  for j in range(NCHUNKS):
    gets[j].wait()
    puts[j] = pltpu.async_copy(bufs[j % 2], out_hbm.at[dest_v.at[j]], sem_s)
    if j + 2 < NCHUNKS:
      puts[j].wait()
      gets[j + 2] = pltpu.async_copy(pf_hbm.at[perm_v.at[j + 2]],
                                     bufs[j % 2], sem_g)
  puts[NCHUNKS - 2].wait()
  puts[NCHUNKS - 1].wait()


def _tr_body(canvas_ref, out_ref):
  out_ref[...] = jnp.swapaxes(canvas_ref[...], 0, 1)[None]


@jax.jit
def kernel(voxels, coords, voxel_num_points, W, gamma, beta):
  pad_c = ((0, 0), (0, C_PAD - C_OUT))
  w1t = jnp.pad(jnp.transpose(W[:, 0:C_PT]), pad_c)
  w2t = jnp.pad(jnp.transpose(W[:, C_PT:C_PT + 3]), pad_c)
  w3t = jnp.pad(jnp.transpose(W[:, C_PT + 3:C_PT + 6]), pad_c)
  w23t = w2t + w3t
  numpf = voxel_num_points.astype(jnp.float32)[:, None]

  maxh, stats = pl.pallas_call(
      _a1_body,
      grid=(NSTEPS,),
      in_specs=[
          pl.BlockSpec((NB, MAX_PTS, C_PT), lambda i: (i, 0, 0)),
          pl.BlockSpec((NB, 4), lambda i: (i, 0)),
          pl.BlockSpec((NB, 1), lambda i: (i, 0)),
          pl.BlockSpec((C_PT, C_PAD), lambda i: (0, 0)),
          pl.BlockSpec((3, C_PAD), lambda i: (0, 0)),
          pl.BlockSpec((3, C_PAD), lambda i: (0, 0)),
          pl.BlockSpec((3, C_PAD), lambda i: (0, 0)),
      ],
      out_specs=[
          pl.BlockSpec((NB, C_PAD), lambda i: (i, 0)),
          pl.BlockSpec((8, C_PAD), lambda i: (0, 0)),
      ],
      out_shape=[
          jax.ShapeDtypeStruct((N_VOX, C_PAD), jnp.float32),
          jax.ShapeDtypeStruct((8, C_PAD), jnp.float32),
      ],
  )(voxels, coords, numpf, w1t, w23t, w2t, w3t)

  cnt = float(N_VOX * MAX_PTS)
  mu = stats[0, 0:C_OUT] / cnt
  var = stats[1, 0:C_OUT] / cnt - mu * mu
  scale = gamma * lax.rsqrt(var + EPS)
  shift = beta - mu * scale

  pf = pl.pallas_call(
      _a2_body,
      grid=(4,),
      in_specs=[
          pl.BlockSpec((N_VOX // 4, C_PAD), lambda i: (i, 0)),
          pl.BlockSpec((1, C_OUT), lambda i: (0, 0)),
          pl.BlockSpec((1, C_OUT), lambda i: (0, 0)),
      ],
      out_specs=pl.BlockSpec((N_VOX // 4, C_OUT), lambda i: (i, 0)),
      out_shape=jax.ShapeDtypeStruct((N_VOX, C_OUT), jnp.float32),
  )(maxh, scale[None, :], shift[None, :])

  key = coords[:, 0] * CELLS + coords[:, 2] * NX + coords[:, 3]
  sk, perm = lax.sort_key_val(key, jnp.arange(N_VOX, dtype=jnp.int32),
                              is_stable=True)
  is_win = jnp.concatenate(
      [sk[1:] != sk[:-1], jnp.ones((1,), bool)])
  dest = jnp.where(is_win, sk, TRASH)
  pad = PAD_N - N_VOX
  perm_pad = jnp.concatenate(
      [perm, jnp.zeros((pad,), jnp.int32)]).reshape(SC_TILES, NCHUNKS, CHUNK)
  dest_pad = jnp.concatenate(
      [dest, jnp.full((pad,), TRASH, jnp.int32)]).reshape(
          SC_TILES, NCHUNKS, CHUNK)
  zsrc = jnp.zeros((ZCH, C_OUT), jnp.float32)

  mesh = plsc.VectorSubcoreMesh(core_axis_name="c", subcore_axis_name="s",
                                num_cores=SC_CORES, num_subcores=SC_TILES)
  canvas = pl.kernel(
      _sc_body,
      out_type=jax.ShapeDtypeStruct((R_CANVAS, C_OUT), jnp.float32),
      mesh=mesh,
      scratch_types=[
          pltpu.VMEM((ZCH, C_OUT), jnp.float32),
          pltpu.VMEM((NCHUNKS, CHUNK), jnp.int32),
          pltpu.VMEM((NCHUNKS, CHUNK), jnp.int32),
          pltpu.VMEM((CHUNK, C_OUT), jnp.float32),
          pltpu.VMEM((CHUNK, C_OUT), jnp.float32),
          pltpu.SemaphoreType.DMA,
          pltpu.SemaphoreType.DMA,
          pltpu.SemaphoreType.DMA,
      ],
      compiler_params=pltpu.CompilerParams(use_tc_tiling_on_sc=False),
  )(pf, perm_pad, dest_pad, zsrc)

  out = pl.pallas_call(
      _tr_body,
      grid=(BATCH, TBLK),
      in_specs=[pl.BlockSpec((TCOLS, C_OUT), lambda b, j: (b * TBLK + j, 0))],
      out_specs=pl.BlockSpec((1, C_OUT, TCOLS), lambda b, j: (b, 0, j)),
      out_shape=jax.ShapeDtypeStruct((BATCH, C_OUT, CELLS), jnp.float32),
  )(canvas)
  return out.reshape(BATCH, C_OUT * NZ, NY, NX)
